# baseline (speedup 1.0000x reference)
import functools

import numpy as np
import jax
import jax.numpy as jnp
from jax import lax
from jax.experimental import pallas as pl
from jax.experimental.pallas import tpu as pltpu
from jax.experimental.pallas import tpu_sc as plsc

C = 1024
H = 16
POS = 16
S = 33
SP = 48
A = 17
BL = 256


def _silu(v):
    return v * jax.nn.sigmoid(v)


def _np_consts():
    g = np.zeros((H * POS, H), np.float32)
    g[np.arange(H * POS), np.arange(H * POS) // POS] = 1.0
    r = g.T.copy()
    p_l = np.zeros((A * H, H * SP), np.float32)
    p_r = np.zeros((A * H, H * SP), np.float32)
    for a in range(A):
        for h in range(H):
            if a >= 1:
                p_l[a * H + h, h * SP + (16 - a)] = 1.0
            p_r[a * H + h, h * SP + (16 + a)] = 1.0
    e_l = np.zeros((SP, A * H), np.float32)
    e_r = np.zeros((SP, A * H), np.float32)
    for a in range(A):
        if a >= 1:
            e_l[16 - a, a * H:(a + 1) * H] = 1.0
        e_r[16 + a, a * H:(a + 1) * H] = 1.0
    return g, r, p_l, p_r, e_l, e_r


_G, _R, _PL, _PR, _EL, _ER = _np_consts()


_DNT = (((1,), (1,)), ((), ()))


def _stage1_math(i, r0, L, xb, Ww, bw, Wq, bq, wk_row, g, r, e_l, e_r, p_l, p_r):
    uw = lax.dot_general(xb, Ww, _DNT, preferred_element_type=jnp.float32) + bw
    w0 = _silu(uw[:, 0:H])
    w1 = _silu(uw[:, H:2 * H])
    w2 = _silu(uw[:, 2 * H:3 * H])
    q2 = _silu(lax.dot_general(xb, Wq, _DNT, preferred_element_type=jnp.float32) + bq)
    freq = jax.nn.sigmoid(w0) * 15.0 + 1.0
    phase = jnp.tanh(w1) * 16.0
    decay = jax.nn.sigmoid(w2) * 9.5 + 0.5
    freq_avg = jnp.mean(freq, axis=1, keepdims=True)
    phase_avg = jnp.mean(phase, axis=1, keepdims=True)
    centers = (jax.lax.broadcasted_iota(jnp.int32, (BL, 1), 0) + (i * BL + r0)).astype(jnp.float32)
    sgrid = jax.lax.broadcasted_iota(jnp.int32, (BL, SP), 1).astype(jnp.float32) - 16.0
    sp = (centers + sgrid * freq_avg) + phase_avg
    validf = ((sp >= 0) & (sp < L) & (sgrid <= 16.0)).astype(jnp.float32)
    idx = jnp.clip(sp.astype(jnp.int32), 0, L - 1)
    lint = jax.lax.broadcasted_iota(jnp.int32, (BL, SP), 0) + (i * BL + r0)
    idx_safe = jnp.where(validf > 0, idx, lint)
    vL = jnp.dot(validf, e_l, preferred_element_type=jnp.float32)
    vR = jnp.dot(validf, e_r, preferred_element_type=jnp.float32)
    cnt = vL + vR
    freq_rep = jnp.dot(freq, r, preferred_element_type=jnp.float32)
    wk_freq = freq_rep * wk_row
    avec = (jax.lax.broadcasted_iota(jnp.int32, (1, A * H), 1) // H).astype(jnp.float32)
    freq_w = jnp.dot(freq, jnp.tile(jnp.eye(H, dtype=jnp.float32), (1, A)),
                     preferred_element_type=jnp.float32)
    decay_w = jnp.dot(decay, jnp.tile(jnp.eye(H, dtype=jnp.float32), (1, A)),
                      preferred_element_type=jnp.float32)
    env_w = jnp.exp(-(avec * freq_w) / decay_w)
    logits = [jnp.zeros((BL, H), jnp.float32)]
    m = jnp.where(cnt[:, 0:H] > 0, 0.0, -1e30)
    for a in range(1, A):
        keys = _silu(float(a) * wk_freq)
        prod = q2 * keys
        logit_a = jnp.dot(prod, g, preferred_element_type=jnp.float32) * (POS ** -0.5)
        cnt_a = cnt[:, a * H:(a + 1) * H]
        m = jnp.maximum(m, jnp.where(cnt_a > 0, logit_a, -1e30))
        logits.append(logit_a)
    Dn = jnp.zeros((BL, H), jnp.float32)
    Z = jnp.zeros((BL, H), jnp.float32)
    nums = []
    for a in range(A):
        e_a = jnp.exp(logits[a] - m)
        num_a = e_a * env_w[:, a * H:(a + 1) * H]
        cnt_a = cnt[:, a * H:(a + 1) * H]
        Dn = Dn + num_a * cnt_a
        Z = Z + e_a * cnt_a
        nums.append(num_a)
    denom = Dn + 1e-8 * Z
    w17 = jnp.concatenate([n / denom for n in nums], axis=1)
    wf = (jnp.dot(w17 * vL, p_l, preferred_element_type=jnp.float32)
          + jnp.dot(w17 * vR, p_r, preferred_element_type=jnp.float32))
    return idx_safe, wf


def _make_stage1_body(r0, Lfull):
    def _stage1_body(x_ref, ww_ref, bw_ref, wq_ref, bq_ref, wkrow_ref,
                     g_ref, r_ref, el_ref, er_ref, plm_ref, prm_ref,
                     idx_ref, w_ref):
        i = pl.program_id(0)
        idx_safe, wf = _stage1_math(
            i, r0, Lfull, x_ref[...], ww_ref[...], bw_ref[...], wq_ref[...],
            bq_ref[...], wkrow_ref[...], g_ref[...], r_ref[...], el_ref[...],
            er_ref[...], plm_ref[...], prm_ref[...])
        idx_ref[...] = idx_safe
        w_ref[...] = wf
    return _stage1_body


def _stage3_body(o_ref, w1_ref, b1_ref, w2_ref, b2_ref, wo_ref, y_ref):
    o = o_ref[...]
    h1 = _silu(lax.dot_general(o, w1_ref[...], _DNT,
                               preferred_element_type=jnp.float32) + b1_ref[...])
    se = jax.nn.sigmoid(lax.dot_general(h1, w2_ref[...], _DNT,
                                        preferred_element_type=jnp.float32) + b2_ref[...])
    o2 = o * se
    y = lax.dot_general(o2, wo_ref[...], _DNT,
                        preferred_element_type=jnp.float32)
    y_ref[...] = _silu(y)


def _stage1(x2, Ww, bw, Wq, bq, Wk, r0, Lfull):
    L = x2.shape[0]
    wk_row = jnp.tile(Wk[:, 0], (H,))[None, :]
    full = lambda shape: pl.BlockSpec(shape, lambda i: tuple(0 for _ in shape))
    return pl.pallas_call(
        _make_stage1_body(r0, Lfull),
        grid=(L // BL,),
        in_specs=[
            pl.BlockSpec((BL, C), lambda i: (i, 0)),
            full((3 * H, C)), full((1, 3 * H)),
            full((H * POS, C)), full((1, H * POS)),
            full((1, H * POS)),
            full((H * POS, H)), full((H, H * POS)),
            full((SP, A * H)), full((SP, A * H)),
            full((A * H, H * SP)), full((A * H, H * SP)),
        ],
        out_specs=[
            pl.BlockSpec((BL, SP), lambda i: (i, 0)),
            pl.BlockSpec((BL, H * SP), lambda i: (i, 0)),
        ],
        out_shape=[
            jax.ShapeDtypeStruct((L, SP), jnp.int32),
            jax.ShapeDtypeStruct((L, H * SP), jnp.float32),
        ],
    )(x2, Ww, bw[None, :], Wq, bq[None, :], wk_row,
      jnp.asarray(_G), jnp.asarray(_R), jnp.asarray(_EL), jnp.asarray(_ER),
      jnp.asarray(_PL), jnp.asarray(_PR))


BLK = 128
W_WIN = 704
D = C // H


def _stage2(x2, idx48, wflat, r0):
    L = x2.shape[0]
    Lc = idx48.shape[0]
    nblk = Lc // BLK
    mesh = plsc.VectorSubcoreMesh(core_axis_name="c", subcore_axis_name="s")

    @functools.partial(
        pl.kernel,
        mesh=mesh,
        compiler_params=pltpu.CompilerParams(
            needs_layout_passes=False, use_tc_tiling_on_sc=False),
        out_type=jax.ShapeDtypeStruct((Lc, C), jnp.float32),
        scratch_types=[
            pltpu.VMEM((W_WIN, D), jnp.float32),
            pltpu.VMEM((W_WIN, D), jnp.float32),
            pltpu.VMEM((BLK, SP), jnp.int32),
            pltpu.VMEM((BLK, SP), jnp.float32),
            pltpu.VMEM((BLK, D), jnp.float32),
            pltpu.SemaphoreType.DMA,
            pltpu.SemaphoreType.DMA,
        ],
    )
    def sc_body(x_hbm, idx_hbm, w_hbm, out_hbm, xw0_v, xw1_v, idx_v, w_v,
                out_v, sem0, sem1):
        wid = lax.axis_index("s") * 2 + lax.axis_index("c")
        h = wid % H
        half = wid // H
        nk = nblk // 2
        bufs = (xw0_v, xw1_v)
        sems = (sem0, sem1)

        def win_start(k):
            ws = pl.multiple_of(
                jnp.clip(r0 + (half * nk + k) * BLK - 288, 0, L - W_WIN), 8)
            cp = pltpu.async_copy(
                x_hbm.at[pl.ds(ws, W_WIN), pl.ds(h * D, D)],
                bufs[k % 2], sems[k % 2])
            return ws, cp

        pend = win_start(0)
        for k in range(nk):
            blk = half * nk + k
            l0 = pl.multiple_of(blk * BLK, BLK)
            ws, cp = pend
            pltpu.sync_copy(idx_hbm.at[pl.ds(l0, BLK), :], idx_v)
            pltpu.sync_copy(w_hbm.at[pl.ds(l0, BLK), pl.ds(h * SP, SP)], w_v)
            cp.wait()
            if k + 1 < nk:
                pend = win_start(k + 1)
            xw_v = bufs[k % 2]

            def body(i, carry):
                for l in (4 * i, 4 * i + 1, 4 * i + 2, 4 * i + 3):
                    wrows = [w_v[l, pl.ds(16 * t, 16)] for t in range(SP // 16)]
                    irows = [idx_v[l, pl.ds(16 * t, 16)] - ws
                             for t in range(SP // 16)]
                    accs = [jnp.zeros((16,), jnp.float32)
                            for _ in range(D // 16)]
                    for s in range(S):
                        wvec = jnp.full((16,), wrows[s // 16][s % 16],
                                        jnp.float32)
                        irow = irows[s // 16][s % 16]
                        for j in range(D // 16):
                            vals = xw_v[irow, pl.ds(16 * j, 16)]
                            accs[j] = accs[j] + wvec * vals
                    for j in range(D // 16):
                        out_v[l, pl.ds(16 * j, 16)] = accs[j]
                return carry

            lax.fori_loop(0, BLK // 4, body, 0)
            pltpu.sync_copy(out_v, out_hbm.at[pl.ds(l0, BLK), pl.ds(h * D, D)])

    return sc_body(x2, idx48, wflat)


def _stage3(out, W1, b1, W2, b2, Wo):
    L = out.shape[0]
    full = lambda shape: pl.BlockSpec(shape, lambda i: tuple(0 for _ in shape))
    return pl.pallas_call(
        _stage3_body,
        grid=(L // BL,),
        in_specs=[
            pl.BlockSpec((BL, C), lambda i: (i, 0)),
            full((C // 4, C)), full((1, C // 4)),
            full((C, C // 4)), full((1, C)),
            full((C, C)),
        ],
        out_specs=pl.BlockSpec((BL, C), lambda i: (i, 0)),
        out_shape=jax.ShapeDtypeStruct((L, C), jnp.float32),
    )(out, W1, b1[None, :], W2, b2[None, :], Wo)


def kernel(x, Ww, bw, Wq, bq, Wk, Wo, W1, b1, W2, b2):
    B, L, _ = x.shape
    x2 = x.reshape(L, C)
    idx48, wflat = _stage1(x2, Ww, bw, Wq, bq, Wk, 0, L)
    out = _stage2(x2, idx48, wflat, 0)
    y = _stage3(out, W1, b1, W2, b2, Wo)
    return y.reshape(B, L, C)


# speedup vs baseline: 1.0350x; 1.0350x over previous
import functools

import numpy as np
import jax
import jax.numpy as jnp
from jax import lax
from jax.experimental import pallas as pl
from jax.experimental.pallas import tpu as pltpu
from jax.experimental.pallas import tpu_sc as plsc

C = 1024
H = 16
POS = 16
S = 33
SP = 48
A = 17
BL = 256


def _silu(v):
    return v * jax.nn.sigmoid(v)


def _np_consts():
    g = np.zeros((H * POS, H), np.float32)
    g[np.arange(H * POS), np.arange(H * POS) // POS] = 1.0
    r = g.T.copy()
    p_l = np.zeros((A * H, H * SP), np.float32)
    p_r = np.zeros((A * H, H * SP), np.float32)
    for a in range(A):
        for h in range(H):
            if a >= 1:
                p_l[a * H + h, h * SP + (16 - a)] = 1.0
            p_r[a * H + h, h * SP + (16 + a)] = 1.0
    e_l = np.zeros((SP, A * H), np.float32)
    e_r = np.zeros((SP, A * H), np.float32)
    for a in range(A):
        if a >= 1:
            e_l[16 - a, a * H:(a + 1) * H] = 1.0
        e_r[16 + a, a * H:(a + 1) * H] = 1.0
    return g, r, p_l, p_r, e_l, e_r


_G, _R, _PL, _PR, _EL, _ER = _np_consts()


_DNT = (((1,), (1,)), ((), ()))


def _stage1_math(i, r0, L, xb, Ww, bw, Wq, bq, wk_row, g, r, e_l, e_r, p_l, p_r):
    uw = lax.dot_general(xb, Ww, _DNT, preferred_element_type=jnp.float32) + bw
    w0 = _silu(uw[:, 0:H])
    w1 = _silu(uw[:, H:2 * H])
    w2 = _silu(uw[:, 2 * H:3 * H])
    q2 = _silu(lax.dot_general(xb, Wq, _DNT, preferred_element_type=jnp.float32) + bq)
    freq = jax.nn.sigmoid(w0) * 15.0 + 1.0
    phase = jnp.tanh(w1) * 16.0
    decay = jax.nn.sigmoid(w2) * 9.5 + 0.5
    freq_avg = jnp.mean(freq, axis=1, keepdims=True)
    phase_avg = jnp.mean(phase, axis=1, keepdims=True)
    centers = (jax.lax.broadcasted_iota(jnp.int32, (BL, 1), 0) + (i * BL + r0)).astype(jnp.float32)
    sgrid = jax.lax.broadcasted_iota(jnp.int32, (BL, SP), 1).astype(jnp.float32) - 16.0
    sp = (centers + sgrid * freq_avg) + phase_avg
    validf = ((sp >= 0) & (sp < L) & (sgrid <= 16.0)).astype(jnp.float32)
    idx = jnp.clip(sp.astype(jnp.int32), 0, L - 1)
    lint = jax.lax.broadcasted_iota(jnp.int32, (BL, SP), 0) + (i * BL + r0)
    idx_safe = jnp.where(validf > 0, idx, lint)
    vL = jnp.dot(validf, e_l, preferred_element_type=jnp.float32)
    vR = jnp.dot(validf, e_r, preferred_element_type=jnp.float32)
    cnt = vL + vR
    freq_rep = jnp.dot(freq, r, preferred_element_type=jnp.float32)
    wk_freq = freq_rep * wk_row
    avec = (jax.lax.broadcasted_iota(jnp.int32, (1, A * H), 1) // H).astype(jnp.float32)
    freq_w = jnp.dot(freq, jnp.tile(jnp.eye(H, dtype=jnp.float32), (1, A)),
                     preferred_element_type=jnp.float32)
    decay_w = jnp.dot(decay, jnp.tile(jnp.eye(H, dtype=jnp.float32), (1, A)),
                      preferred_element_type=jnp.float32)
    env_w = jnp.exp(-(avec * freq_w) / decay_w)
    logits = [jnp.zeros((BL, H), jnp.float32)]
    m = jnp.where(cnt[:, 0:H] > 0, 0.0, -1e30)
    for a in range(1, A):
        keys = _silu(float(a) * wk_freq)
        prod = q2 * keys
        logit_a = jnp.dot(prod, g, preferred_element_type=jnp.float32) * (POS ** -0.5)
        cnt_a = cnt[:, a * H:(a + 1) * H]
        m = jnp.maximum(m, jnp.where(cnt_a > 0, logit_a, -1e30))
        logits.append(logit_a)
    Dn = jnp.zeros((BL, H), jnp.float32)
    Z = jnp.zeros((BL, H), jnp.float32)
    nums = []
    for a in range(A):
        e_a = jnp.exp(logits[a] - m)
        num_a = e_a * env_w[:, a * H:(a + 1) * H]
        cnt_a = cnt[:, a * H:(a + 1) * H]
        Dn = Dn + num_a * cnt_a
        Z = Z + e_a * cnt_a
        nums.append(num_a)
    denom = Dn + 1e-8 * Z
    w17 = jnp.concatenate([n / denom for n in nums], axis=1)
    wf = (jnp.dot((w17 * vL).astype(jnp.bfloat16), p_l,
                  preferred_element_type=jnp.float32)
          + jnp.dot((w17 * vR).astype(jnp.bfloat16), p_r,
                    preferred_element_type=jnp.float32))
    return idx_safe, wf


def _make_stage1_body(r0, Lfull):
    def _stage1_body(x_ref, ww_ref, bw_ref, wq_ref, bq_ref, wkrow_ref,
                     g_ref, r_ref, el_ref, er_ref, plm_ref, prm_ref,
                     idx_ref, w_ref):
        i = pl.program_id(0)
        idx_safe, wf = _stage1_math(
            i, r0, Lfull, x_ref[...], ww_ref[...], bw_ref[...], wq_ref[...],
            bq_ref[...], wkrow_ref[...], g_ref[...], r_ref[...], el_ref[...],
            er_ref[...], plm_ref[...], prm_ref[...])
        idx_ref[...] = idx_safe
        w_ref[...] = wf
    return _stage1_body


def _stage3_body(o_ref, w1_ref, b1_ref, w2_ref, b2_ref, wo_ref, y_ref):
    o = o_ref[...]
    ob = o.astype(jnp.bfloat16)
    h1 = _silu(lax.dot_general(ob, w1_ref[...], _DNT,
                               preferred_element_type=jnp.float32) + b1_ref[...])
    se = jax.nn.sigmoid(lax.dot_general(h1.astype(jnp.bfloat16), w2_ref[...], _DNT,
                                        preferred_element_type=jnp.float32) + b2_ref[...])
    o2 = (o * se).astype(jnp.bfloat16)
    y = lax.dot_general(o2, wo_ref[...], _DNT,
                        preferred_element_type=jnp.float32)
    y_ref[...] = _silu(y)


def _stage1(x2, Ww, bw, Wq, bq, Wk, r0, Lfull):
    L = x2.shape[0]
    wk_row = jnp.tile(Wk[:, 0], (H,))[None, :]
    full = lambda shape: pl.BlockSpec(shape, lambda i: tuple(0 for _ in shape))
    return pl.pallas_call(
        _make_stage1_body(r0, Lfull),
        grid=(L // BL,),
        in_specs=[
            pl.BlockSpec((BL, C), lambda i: (i, 0)),
            full((3 * H, C)), full((1, 3 * H)),
            full((H * POS, C)), full((1, H * POS)),
            full((1, H * POS)),
            full((H * POS, H)), full((H, H * POS)),
            full((SP, A * H)), full((SP, A * H)),
            full((A * H, H * SP)), full((A * H, H * SP)),
        ],
        out_specs=[
            pl.BlockSpec((BL, SP), lambda i: (i, 0)),
            pl.BlockSpec((BL, H * SP), lambda i: (i, 0)),
        ],
        out_shape=[
            jax.ShapeDtypeStruct((L, SP), jnp.int32),
            jax.ShapeDtypeStruct((L, H * SP), jnp.float32),
        ],
    )(x2, Ww, bw[None, :], Wq, bq[None, :], wk_row,
      jnp.asarray(_G), jnp.asarray(_R), jnp.asarray(_EL), jnp.asarray(_ER),
      jnp.asarray(_PL, dtype=jnp.bfloat16), jnp.asarray(_PR, dtype=jnp.bfloat16))


BLK = 128
W_WIN = 704
D = C // H


def _stage2(x2, idx48, wflat, r0):
    L = x2.shape[0]
    Lc = idx48.shape[0]
    nblk = Lc // BLK
    mesh = plsc.VectorSubcoreMesh(core_axis_name="c", subcore_axis_name="s")

    @functools.partial(
        pl.kernel,
        mesh=mesh,
        compiler_params=pltpu.CompilerParams(
            needs_layout_passes=False, use_tc_tiling_on_sc=False),
        out_type=jax.ShapeDtypeStruct((Lc, C), jnp.float32),
        scratch_types=[
            pltpu.VMEM((W_WIN, D), jnp.float32),
            pltpu.VMEM((W_WIN, D), jnp.float32),
            pltpu.VMEM((BLK, SP), jnp.int32),
            pltpu.VMEM((BLK, SP), jnp.float32),
            pltpu.VMEM((BLK, D), jnp.float32),
            pltpu.SemaphoreType.DMA,
            pltpu.SemaphoreType.DMA,
        ],
    )
    def sc_body(x_hbm, idx_hbm, w_hbm, out_hbm, xw0_v, xw1_v, idx_v, w_v,
                out_v, sem0, sem1):
        wid = lax.axis_index("s") * 2 + lax.axis_index("c")
        h = wid % H
        half = wid // H
        nk = nblk // 2
        bufs = (xw0_v, xw1_v)
        sems = (sem0, sem1)

        def win_start(k):
            ws = pl.multiple_of(
                jnp.clip(r0 + (half * nk + k) * BLK - 288, 0, L - W_WIN), 8)
            cp = pltpu.async_copy(
                x_hbm.at[pl.ds(ws, W_WIN), pl.ds(h * D, D)],
                bufs[k % 2], sems[k % 2])
            return ws, cp

        pend = win_start(0)
        for k in range(nk):
            blk = half * nk + k
            l0 = pl.multiple_of(blk * BLK, BLK)
            ws, cp = pend
            pltpu.sync_copy(idx_hbm.at[pl.ds(l0, BLK), :], idx_v)
            pltpu.sync_copy(w_hbm.at[pl.ds(l0, BLK), pl.ds(h * SP, SP)], w_v)
            cp.wait()
            if k + 1 < nk:
                pend = win_start(k + 1)
            xw_v = bufs[k % 2]

            def body(i, carry):
                for l in (2 * i, 2 * i + 1):
                    wrows = [w_v[l, pl.ds(16 * t, 16)] for t in range(SP // 16)]
                    irows = [idx_v[l, pl.ds(16 * t, 16)] - ws
                             for t in range(SP // 16)]
                    accs = [jnp.zeros((16,), jnp.float32)
                            for _ in range(D // 16)]
                    for s in range(S):
                        wvec = jnp.full((16,), wrows[s // 16][s % 16],
                                        jnp.float32)
                        irow = irows[s // 16][s % 16]
                        for j in range(D // 16):
                            vals = xw_v[irow, pl.ds(16 * j, 16)]
                            accs[j] = accs[j] + wvec * vals
                    for j in range(D // 16):
                        out_v[l, pl.ds(16 * j, 16)] = accs[j]
                return carry

            lax.fori_loop(0, BLK // 2, body, 0)
            pltpu.sync_copy(out_v, out_hbm.at[pl.ds(l0, BLK), pl.ds(h * D, D)])

    return sc_body(x2, idx48, wflat)


def _stage3(out, W1, b1, W2, b2, Wo):
    L = out.shape[0]
    full = lambda shape: pl.BlockSpec(shape, lambda i: tuple(0 for _ in shape))
    return pl.pallas_call(
        _stage3_body,
        grid=(L // BL,),
        in_specs=[
            pl.BlockSpec((BL, C), lambda i: (i, 0)),
            full((C // 4, C)), full((1, C // 4)),
            full((C, C // 4)), full((1, C)),
            full((C, C)),
        ],
        out_specs=pl.BlockSpec((BL, C), lambda i: (i, 0)),
        out_shape=jax.ShapeDtypeStruct((L, C), jnp.float32),
    )(out, W1.astype(jnp.bfloat16), b1[None, :],
      W2.astype(jnp.bfloat16), b2[None, :], Wo.astype(jnp.bfloat16))


def kernel(x, Ww, bw, Wq, bq, Wk, Wo, W1, b1, W2, b2):
    B, L, _ = x.shape
    x2 = x.reshape(L, C)
    idx48, wflat = _stage1(x2, Ww, bw, Wq, bq, Wk, 0, L)
    out = _stage2(x2, idx48, wflat, 0)
    y = _stage3(out, W1, b1, W2, b2, Wo)
    return y.reshape(B, L, C)


# speedup vs baseline: 1.0448x; 1.0094x over previous
import functools

import numpy as np
import jax
import jax.numpy as jnp
from jax import lax
from jax.experimental import pallas as pl
from jax.experimental.pallas import tpu as pltpu
from jax.experimental.pallas import tpu_sc as plsc

C = 1024
H = 16
POS = 16
S = 33
SP = 48
A = 17
BL = 256


def _silu(v):
    return v * jax.nn.sigmoid(v)


def _np_consts():
    g = np.zeros((H * POS, H), np.float32)
    g[np.arange(H * POS), np.arange(H * POS) // POS] = 1.0
    r = g.T.copy()
    p_l = np.zeros((A * H, H * SP), np.float32)
    p_r = np.zeros((A * H, H * SP), np.float32)
    for a in range(A):
        for h in range(H):
            if a >= 1:
                p_l[a * H + h, h * SP + (16 - a)] = 1.0
            p_r[a * H + h, h * SP + (16 + a)] = 1.0
    e_l = np.zeros((SP, A * H), np.float32)
    e_r = np.zeros((SP, A * H), np.float32)
    for a in range(A):
        if a >= 1:
            e_l[16 - a, a * H:(a + 1) * H] = 1.0
        e_r[16 + a, a * H:(a + 1) * H] = 1.0
    return g, r, p_l, p_r, e_l, e_r


_G, _R, _PL, _PR, _EL, _ER = _np_consts()


_DNT = (((1,), (1,)), ((), ()))


def _stage1_math(i, r0, L, xb, Ww, bw, Wq, bq, wk_row, g, r, e_l, e_r, p_l, p_r):
    uw = lax.dot_general(xb, Ww, _DNT, preferred_element_type=jnp.float32) + bw
    w0 = _silu(uw[:, 0:H])
    w1 = _silu(uw[:, H:2 * H])
    w2 = _silu(uw[:, 2 * H:3 * H])
    q2 = _silu(lax.dot_general(xb, Wq, _DNT, preferred_element_type=jnp.float32) + bq)
    freq = jax.nn.sigmoid(w0) * 15.0 + 1.0
    phase = jnp.tanh(w1) * 16.0
    decay = jax.nn.sigmoid(w2) * 9.5 + 0.5
    freq_avg = jnp.mean(freq, axis=1, keepdims=True)
    phase_avg = jnp.mean(phase, axis=1, keepdims=True)
    centers = (jax.lax.broadcasted_iota(jnp.int32, (BL, 1), 0) + (i * BL + r0)).astype(jnp.float32)
    sgrid = jax.lax.broadcasted_iota(jnp.int32, (BL, SP), 1).astype(jnp.float32) - 16.0
    sp = (centers + sgrid * freq_avg) + phase_avg
    validf = ((sp >= 0) & (sp < L) & (sgrid <= 16.0)).astype(jnp.float32)
    idx = jnp.clip(sp.astype(jnp.int32), 0, L - 1)
    lint = jax.lax.broadcasted_iota(jnp.int32, (BL, SP), 0) + (i * BL + r0)
    idx_safe = jnp.where(validf > 0, idx, lint)
    vL = jnp.dot(validf, e_l, preferred_element_type=jnp.float32)
    vR = jnp.dot(validf, e_r, preferred_element_type=jnp.float32)
    cnt = vL + vR
    freq_rep = jnp.dot(freq, r, preferred_element_type=jnp.float32)
    wk_freq = freq_rep * wk_row
    avec = (jax.lax.broadcasted_iota(jnp.int32, (1, A * H), 1) // H).astype(jnp.float32)
    freq_w = jnp.dot(freq, jnp.tile(jnp.eye(H, dtype=jnp.float32), (1, A)),
                     preferred_element_type=jnp.float32)
    decay_w = jnp.dot(decay, jnp.tile(jnp.eye(H, dtype=jnp.float32), (1, A)),
                      preferred_element_type=jnp.float32)
    env_w = jnp.exp(-(avec * freq_w) / decay_w)
    logits = [jnp.zeros((BL, H), jnp.float32)]
    m = jnp.where(cnt[:, 0:H] > 0, 0.0, -1e30)
    u = jnp.exp(-wk_freq)
    p = u
    for a in range(1, A):
        keys = (float(a) * wk_freq) / (1.0 + p)
        prod = q2 * keys
        logit_a = jnp.dot(prod, g, preferred_element_type=jnp.float32) * (POS ** -0.5)
        cnt_a = cnt[:, a * H:(a + 1) * H]
        m = jnp.maximum(m, jnp.where(cnt_a > 0, logit_a, -1e30))
        logits.append(logit_a)
        if a + 1 < A:
            p = p * u
    Dn = jnp.zeros((BL, H), jnp.float32)
    Z = jnp.zeros((BL, H), jnp.float32)
    nums = []
    for a in range(A):
        e_a = jnp.exp(logits[a] - m)
        num_a = e_a * env_w[:, a * H:(a + 1) * H]
        cnt_a = cnt[:, a * H:(a + 1) * H]
        Dn = Dn + num_a * cnt_a
        Z = Z + e_a * cnt_a
        nums.append(num_a)
    denom = Dn + 1e-8 * Z
    w17 = jnp.concatenate([n / denom for n in nums], axis=1)
    wf = (jnp.dot((w17 * vL).astype(jnp.bfloat16), p_l,
                  preferred_element_type=jnp.float32)
          + jnp.dot((w17 * vR).astype(jnp.bfloat16), p_r,
                    preferred_element_type=jnp.float32))
    return idx_safe, wf


def _make_stage1_body(r0, Lfull):
    def _stage1_body(x_ref, ww_ref, bw_ref, wq_ref, bq_ref, wkrow_ref,
                     g_ref, r_ref, el_ref, er_ref, plm_ref, prm_ref,
                     idx_ref, w_ref):
        i = pl.program_id(0)
        idx_safe, wf = _stage1_math(
            i, r0, Lfull, x_ref[...], ww_ref[...], bw_ref[...], wq_ref[...],
            bq_ref[...], wkrow_ref[...], g_ref[...], r_ref[...], el_ref[...],
            er_ref[...], plm_ref[...], prm_ref[...])
        idx_ref[...] = idx_safe
        w_ref[...] = wf
    return _stage1_body


def _stage3_body(o_ref, w1_ref, b1_ref, w2_ref, b2_ref, wo_ref, y_ref):
    o = o_ref[...]
    ob = o.astype(jnp.bfloat16)
    h1 = _silu(lax.dot_general(ob, w1_ref[...], _DNT,
                               preferred_element_type=jnp.float32) + b1_ref[...])
    se = jax.nn.sigmoid(lax.dot_general(h1.astype(jnp.bfloat16), w2_ref[...], _DNT,
                                        preferred_element_type=jnp.float32) + b2_ref[...])
    o2 = (o * se).astype(jnp.bfloat16)
    y = lax.dot_general(o2, wo_ref[...], _DNT,
                        preferred_element_type=jnp.float32)
    y_ref[...] = _silu(y)


def _stage1(x2, Ww, bw, Wq, bq, Wk, r0, Lfull):
    L = x2.shape[0]
    wk_row = jnp.tile(Wk[:, 0], (H,))[None, :]
    full = lambda shape: pl.BlockSpec(shape, lambda i: tuple(0 for _ in shape))
    return pl.pallas_call(
        _make_stage1_body(r0, Lfull),
        grid=(L // BL,),
        in_specs=[
            pl.BlockSpec((BL, C), lambda i: (i, 0)),
            full((3 * H, C)), full((1, 3 * H)),
            full((H * POS, C)), full((1, H * POS)),
            full((1, H * POS)),
            full((H * POS, H)), full((H, H * POS)),
            full((SP, A * H)), full((SP, A * H)),
            full((A * H, H * SP)), full((A * H, H * SP)),
        ],
        out_specs=[
            pl.BlockSpec((BL, SP), lambda i: (i, 0)),
            pl.BlockSpec((BL, H * SP), lambda i: (i, 0)),
        ],
        out_shape=[
            jax.ShapeDtypeStruct((L, SP), jnp.int32),
            jax.ShapeDtypeStruct((L, H * SP), jnp.float32),
        ],
    )(x2, Ww, bw[None, :], Wq, bq[None, :], wk_row,
      jnp.asarray(_G), jnp.asarray(_R), jnp.asarray(_EL), jnp.asarray(_ER),
      jnp.asarray(_PL, dtype=jnp.bfloat16), jnp.asarray(_PR, dtype=jnp.bfloat16))


BLK = 128
W_WIN = 704
D = C // H


def _stage2(x2, idx48, wflat, r0):
    L = x2.shape[0]
    Lc = idx48.shape[0]
    nblk = Lc // BLK
    mesh = plsc.VectorSubcoreMesh(core_axis_name="c", subcore_axis_name="s")

    @functools.partial(
        pl.kernel,
        mesh=mesh,
        compiler_params=pltpu.CompilerParams(
            needs_layout_passes=False, use_tc_tiling_on_sc=False),
        out_type=jax.ShapeDtypeStruct((Lc, C), jnp.float32),
        scratch_types=[
            pltpu.VMEM((W_WIN, D), jnp.float32),
            pltpu.VMEM((W_WIN, D), jnp.float32),
            pltpu.VMEM((BLK, SP), jnp.int32),
            pltpu.VMEM((BLK, SP), jnp.float32),
            pltpu.VMEM((BLK, D), jnp.float32),
            pltpu.SemaphoreType.DMA,
            pltpu.SemaphoreType.DMA,
        ],
    )
    def sc_body(x_hbm, idx_hbm, w_hbm, out_hbm, xw0_v, xw1_v, idx_v, w_v,
                out_v, sem0, sem1):
        wid = lax.axis_index("s") * 2 + lax.axis_index("c")
        h = wid % H
        half = wid // H
        nk = nblk // 2
        bufs = (xw0_v, xw1_v)
        sems = (sem0, sem1)

        def win_start(k):
            ws = pl.multiple_of(
                jnp.clip(r0 + (half * nk + k) * BLK - 288, 0, L - W_WIN), 8)
            cp = pltpu.async_copy(
                x_hbm.at[pl.ds(ws, W_WIN), pl.ds(h * D, D)],
                bufs[k % 2], sems[k % 2])
            return ws, cp

        pend = win_start(0)
        for k in range(nk):
            blk = half * nk + k
            l0 = pl.multiple_of(blk * BLK, BLK)
            ws, cp = pend
            pltpu.sync_copy(idx_hbm.at[pl.ds(l0, BLK), :], idx_v)
            pltpu.sync_copy(w_hbm.at[pl.ds(l0, BLK), pl.ds(h * SP, SP)], w_v)
            cp.wait()
            if k + 1 < nk:
                pend = win_start(k + 1)
            xw_v = bufs[k % 2]

            def body(i, carry):
                for l in (2 * i, 2 * i + 1):
                    wrows = [w_v[l, pl.ds(16 * t, 16)] for t in range(SP // 16)]
                    irows = [idx_v[l, pl.ds(16 * t, 16)] - ws
                             for t in range(SP // 16)]
                    accs = [jnp.zeros((16,), jnp.float32)
                            for _ in range(D // 16)]
                    for s in range(S):
                        wvec = jnp.full((16,), wrows[s // 16][s % 16],
                                        jnp.float32)
                        irow = irows[s // 16][s % 16]
                        for j in range(D // 16):
                            vals = xw_v[irow, pl.ds(16 * j, 16)]
                            accs[j] = accs[j] + wvec * vals
                    for j in range(D // 16):
                        out_v[l, pl.ds(16 * j, 16)] = accs[j]
                return carry

            lax.fori_loop(0, BLK // 2, body, 0)
            pltpu.sync_copy(out_v, out_hbm.at[pl.ds(l0, BLK), pl.ds(h * D, D)])

    return sc_body(x2, idx48, wflat)


def _stage3(out, W1, b1, W2, b2, Wo):
    L = out.shape[0]
    full = lambda shape: pl.BlockSpec(shape, lambda i: tuple(0 for _ in shape))
    return pl.pallas_call(
        _stage3_body,
        grid=(L // BL,),
        in_specs=[
            pl.BlockSpec((BL, C), lambda i: (i, 0)),
            full((C // 4, C)), full((1, C // 4)),
            full((C, C // 4)), full((1, C)),
            full((C, C)),
        ],
        out_specs=pl.BlockSpec((BL, C), lambda i: (i, 0)),
        out_shape=jax.ShapeDtypeStruct((L, C), jnp.float32),
    )(out, W1.astype(jnp.bfloat16), b1[None, :],
      W2.astype(jnp.bfloat16), b2[None, :], Wo.astype(jnp.bfloat16))


def kernel(x, Ww, bw, Wq, bq, Wk, Wo, W1, b1, W2, b2):
    B, L, _ = x.shape
    x2 = x.reshape(L, C)
    idx48, wflat = _stage1(x2, Ww, bw, Wq, bq, Wk, 0, L)
    out = _stage2(x2, idx48, wflat, 0)
    y = _stage3(out, W1, b1, W2, b2, Wo)
    return y.reshape(B, L, C)
